# baseline (speedup 1.0000x reference)
import functools

import jax
import jax.numpy as jnp
from jax.experimental import pallas as pl
from jax.experimental.pallas import tpu as pltpu


def _permute_kernel(st_ref, x_ref, xg_ref, *, tm):
    base = pl.program_id(1) * tm

    def body(i, _):
        rows = [x_ref[pl.ds(st_ref[base + 8 * i + j], 1), :] for j in range(8)]
        blk = jnp.concatenate(rows, axis=0).astype(jnp.bfloat16)
        xg_ref[pl.ds(pl.multiple_of(8 * i, 8), 8), :] = blk
        return 0

    jax.lax.fori_loop(0, tm // 8, body, 0, unroll=2)


def _up_kernel(te_ref, xg_ref, up_ref, h_ref):
    acc = jnp.dot(xg_ref[...], up_ref[...].astype(jnp.bfloat16),
                  preferred_element_type=jnp.float32)
    h_ref[...] = (acc * jax.lax.logistic(acc)).astype(jnp.bfloat16)


def _down_kernel(te_ref, rw_ref, h_ref, dw_ref, o_ref):
    acc = jnp.dot(h_ref[...], dw_ref[...].astype(jnp.bfloat16),
                  preferred_element_type=jnp.float32)
    o_ref[...] = acc * rw_ref[...]


def _combine_kernel(pos_ref, o_ref, out_ref, *, tmt, k, kp):
    t0 = pl.program_id(1) * tmt

    def body(g, _):
        rows = []
        for j in range(8):
            base = (t0 + 8 * g + j) * k
            acc = None
            for kk in range(k):
                p = pos_ref[base + kk]
                for p_i in range(kp):
                    v = o_ref[p_i, pl.ds(p, 1), :]
                    acc = v if acc is None else acc + v
            rows.append(acc)
        out_ref[pl.ds(pl.multiple_of(8 * g, 8), 8), :] = jnp.concatenate(rows, 0)
        return 0

    jax.lax.fori_loop(0, tmt // 8, body, 0, unroll=2)


def kernel(x, weights, indices, up_projs, down_projs):
    T, D = x.shape
    _, K = indices.shape
    E, _, F = up_projs.shape
    N = T * K

    TM = 256 if N % 256 == 0 else 8
    NPAD = N + E * TM
    n_tiles = NPAD // TM
    TD = min(1024, D)
    TN1 = min(2048, F)
    TN2 = min(1024, D)
    TK2 = min(4096, F)
    KP = F // TK2
    TD2 = min(256, D)
    TMT = min(512, T)

    flat = indices.reshape(-1).astype(jnp.int32)
    onehot = (flat[:, None] == jnp.arange(E, dtype=jnp.int32)[None, :]).astype(jnp.int32)
    csum = jnp.cumsum(onehot, axis=0)
    counts = csum[-1]
    padded = (counts + TM - 1) // TM * TM
    ends = jnp.cumsum(padded)
    offs = ends - padded
    rank = jnp.take_along_axis(csum, flat[:, None], axis=1)[:, 0] - 1
    pos = (offs[flat] + rank).astype(jnp.int32)
    src_token = (jnp.zeros((NPAD,), jnp.int32)
                 .at[pos].set(jnp.arange(N, dtype=jnp.int32) // K))
    row_w = (jnp.zeros((NPAD, 1), jnp.float32)
             .at[pos, 0].set(weights.reshape(-1).astype(jnp.float32)))
    tile_start = jnp.arange(n_tiles, dtype=jnp.int32) * TM
    tile_expert = jnp.minimum(
        jnp.searchsorted(ends, tile_start, side="right"), E - 1).astype(jnp.int32)

    xg = pl.pallas_call(
        functools.partial(_permute_kernel, tm=TM),
        grid_spec=pltpu.PrefetchScalarGridSpec(
            num_scalar_prefetch=1,
            grid=(D // TD, n_tiles),
            in_specs=[pl.BlockSpec((T, TD), lambda d, m, st: (0, d))],
            out_specs=pl.BlockSpec((TM, TD), lambda d, m, st: (m, d)),
        ),
        out_shape=jax.ShapeDtypeStruct((NPAD, D), jnp.bfloat16),
    )(src_token, x)

    h = pl.pallas_call(
        _up_kernel,
        grid_spec=pltpu.PrefetchScalarGridSpec(
            num_scalar_prefetch=1,
            grid=(F // TN1, n_tiles),
            in_specs=[
                pl.BlockSpec((TM, D), lambda n, m, te: (m, 0)),
                pl.BlockSpec((None, D, TN1), lambda n, m, te: (te[m], 0, n)),
            ],
            out_specs=pl.BlockSpec((TM, TN1), lambda n, m, te: (m, n)),
        ),
        out_shape=jax.ShapeDtypeStruct((NPAD, F), jnp.bfloat16),
        compiler_params=pltpu.CompilerParams(
            dimension_semantics=("arbitrary", "arbitrary")),
    )(tile_expert, xg, up_projs)

    o = pl.pallas_call(
        _down_kernel,
        grid_spec=pltpu.PrefetchScalarGridSpec(
            num_scalar_prefetch=1,
            grid=(D // TN2, KP, n_tiles),
            in_specs=[
                pl.BlockSpec((TM, 1), lambda n, k, m, te: (m, 0)),
                pl.BlockSpec((TM, TK2), lambda n, k, m, te: (m, k)),
                pl.BlockSpec((None, TK2, TN2), lambda n, k, m, te: (te[m], k, n)),
            ],
            out_specs=pl.BlockSpec((None, TM, TN2), lambda n, k, m, te: (k, m, n)),
        ),
        out_shape=jax.ShapeDtypeStruct((KP, NPAD, D), jnp.float32),
        compiler_params=pltpu.CompilerParams(
            dimension_semantics=("arbitrary", "arbitrary", "arbitrary")),
    )(tile_expert, row_w, h, down_projs)

    out = pl.pallas_call(
        functools.partial(_combine_kernel, tmt=TMT, k=K, kp=KP),
        grid_spec=pltpu.PrefetchScalarGridSpec(
            num_scalar_prefetch=1,
            grid=(D // TD2, T // TMT),
            in_specs=[pl.BlockSpec((KP, NPAD, TD2), lambda d, t, ps: (0, 0, d))],
            out_specs=pl.BlockSpec((TMT, TD2), lambda d, t, ps: (t, d)),
        ),
        out_shape=jax.ShapeDtypeStruct((T, D), jnp.float32),
    )(pos, o)

    return out.astype(x.dtype)


# speedup vs baseline: 1.0482x; 1.0482x over previous
import functools

import jax
import jax.numpy as jnp
from jax.experimental import pallas as pl
from jax.experimental.pallas import tpu as pltpu


def _permute_kernel(st_ref, x_ref, xg_ref, *, tm):
    base = pl.program_id(1) * tm

    def body(i, _):
        rows = [x_ref[pl.ds(st_ref[base + 8 * i + j], 1), :] for j in range(8)]
        blk = jnp.concatenate(rows, axis=0).astype(jnp.bfloat16)
        xg_ref[pl.ds(pl.multiple_of(8 * i, 8), 8), :] = blk
        return 0

    jax.lax.fori_loop(0, tm // 8, body, 0, unroll=2)


def _up_kernel(te_ref, xg_ref, up_ref, h_ref):
    acc = jnp.dot(xg_ref[...], up_ref[...].astype(jnp.bfloat16),
                  preferred_element_type=jnp.float32)
    h_ref[...] = (acc * jax.lax.logistic(acc)).astype(jnp.bfloat16)


def _down_kernel(te_ref, rw_ref, h_ref, dw_ref, o_ref):
    acc = jnp.dot(h_ref[...], dw_ref[...].astype(jnp.bfloat16),
                  preferred_element_type=jnp.float32)
    o_ref[...] = acc * rw_ref[...]


def _combine_kernel(pos_ref, o_ref, out_ref, *, tmt, k, kp):
    t0 = pl.program_id(1) * tmt

    def body(g, _):
        rows = []
        for j in range(8):
            base = (t0 + 8 * g + j) * k
            acc = None
            for kk in range(k):
                p = pos_ref[base + kk]
                for p_i in range(kp):
                    v = o_ref[p_i, pl.ds(p, 1), :]
                    acc = v if acc is None else acc + v
            rows.append(acc)
        out_ref[pl.ds(pl.multiple_of(8 * g, 8), 8), :] = jnp.concatenate(rows, 0)
        return 0

    jax.lax.fori_loop(0, tmt // 8, body, 0, unroll=2)


def kernel(x, weights, indices, up_projs, down_projs):
    T, D = x.shape
    _, K = indices.shape
    E, _, F = up_projs.shape
    N = T * K

    TM = 256 if N % 256 == 0 else 8
    NPAD = N + E * TM
    n_tiles = NPAD // TM
    TD = min(1024, D)
    TN1 = min(2048, F)
    TN2 = min(512, D)
    TK2 = F
    KP = F // TK2
    TD2 = min(512, D)
    TMT = min(512, T)

    flat = indices.reshape(-1).astype(jnp.int32)
    onehot = (flat[:, None] == jnp.arange(E, dtype=jnp.int32)[None, :]).astype(jnp.int32)
    csum = jnp.cumsum(onehot, axis=0)
    counts = csum[-1]
    padded = (counts + TM - 1) // TM * TM
    ends = jnp.cumsum(padded)
    offs = ends - padded
    rank = jnp.take_along_axis(csum, flat[:, None], axis=1)[:, 0] - 1
    pos = (offs[flat] + rank).astype(jnp.int32)
    src_token = (jnp.zeros((NPAD,), jnp.int32)
                 .at[pos].set(jnp.arange(N, dtype=jnp.int32) // K))
    row_w = (jnp.zeros((NPAD, 1), jnp.float32)
             .at[pos, 0].set(weights.reshape(-1).astype(jnp.float32)))
    tile_start = jnp.arange(n_tiles, dtype=jnp.int32) * TM
    tile_expert = jnp.minimum(
        jnp.searchsorted(ends, tile_start, side="right"), E - 1).astype(jnp.int32)

    xg = pl.pallas_call(
        functools.partial(_permute_kernel, tm=TM),
        grid_spec=pltpu.PrefetchScalarGridSpec(
            num_scalar_prefetch=1,
            grid=(D // TD, n_tiles),
            in_specs=[pl.BlockSpec((T, TD), lambda d, m, st: (0, d))],
            out_specs=pl.BlockSpec((TM, TD), lambda d, m, st: (m, d)),
        ),
        out_shape=jax.ShapeDtypeStruct((NPAD, D), jnp.bfloat16),
    )(src_token, x)

    h = pl.pallas_call(
        _up_kernel,
        grid_spec=pltpu.PrefetchScalarGridSpec(
            num_scalar_prefetch=1,
            grid=(F // TN1, n_tiles),
            in_specs=[
                pl.BlockSpec((TM, D), lambda n, m, te: (m, 0)),
                pl.BlockSpec((None, D, TN1), lambda n, m, te: (te[m], 0, n)),
            ],
            out_specs=pl.BlockSpec((TM, TN1), lambda n, m, te: (m, n)),
        ),
        out_shape=jax.ShapeDtypeStruct((NPAD, F), jnp.bfloat16),
        compiler_params=pltpu.CompilerParams(
            dimension_semantics=("arbitrary", "arbitrary")),
    )(tile_expert, xg, up_projs)

    o = pl.pallas_call(
        _down_kernel,
        grid_spec=pltpu.PrefetchScalarGridSpec(
            num_scalar_prefetch=1,
            grid=(D // TN2, KP, n_tiles),
            in_specs=[
                pl.BlockSpec((TM, 1), lambda n, k, m, te: (m, 0)),
                pl.BlockSpec((TM, TK2), lambda n, k, m, te: (m, k)),
                pl.BlockSpec((None, TK2, TN2), lambda n, k, m, te: (te[m], k, n)),
            ],
            out_specs=pl.BlockSpec((None, TM, TN2), lambda n, k, m, te: (k, m, n)),
        ),
        out_shape=jax.ShapeDtypeStruct((KP, NPAD, D), jnp.float32),
        compiler_params=pltpu.CompilerParams(
            dimension_semantics=("arbitrary", "arbitrary", "arbitrary")),
    )(tile_expert, row_w, h, down_projs)

    out = pl.pallas_call(
        functools.partial(_combine_kernel, tmt=TMT, k=K, kp=KP),
        grid_spec=pltpu.PrefetchScalarGridSpec(
            num_scalar_prefetch=1,
            grid=(D // TD2, T // TMT),
            in_specs=[pl.BlockSpec((KP, NPAD, TD2), lambda d, t, ps: (0, 0, d))],
            out_specs=pl.BlockSpec((TMT, TD2), lambda d, t, ps: (t, d)),
        ),
        out_shape=jax.ShapeDtypeStruct((T, D), jnp.float32),
    )(pos, o)

    return out.astype(x.dtype)
